# baseline (speedup 1.0000x reference)
import functools

import jax
import jax.numpy as jnp
from jax import lax
from jax.experimental import pallas as pl
from jax.experimental.pallas import tpu as pltpu
from jax.experimental.pallas import tpu_sc as plsc

N = 10000
E = 320000
D = 128
G = 64

NTILES = 32
K = 120
CH = 84
EP = NTILES * CH * K
NP = 10240
RPT = NP // 16
BLK = 256
NB = NP // BLK

_mesh = plsc.VectorSubcoreMesh(core_axis_name="c", subcore_axis_name="s")


@functools.partial(
    pl.kernel,
    out_type=jax.ShapeDtypeStruct((2, NP, 128), jnp.float32),
    mesh=_mesh,
    scratch_types=[
        pltpu.VMEM((2, K), jnp.int32),
        pltpu.VMEM((2, K), jnp.int32),
        pltpu.VMEM((K, 128), jnp.float32),
        pltpu.VMEM_SHARED((NP, 128), jnp.float32),
        pltpu.SemaphoreType.DMA,
        pltpu.SemaphoreType.DMA,
    ],
)
def _deg_kernel(edg_hbm, ones_hbm, zeros_hbm, out_hbm,
                eidx0, eidx1, ones_v, acc, ss0, ss1):
    c = lax.axis_index("c")
    s = lax.axis_index("s")
    t = c * 16 + s
    r0 = s * RPT
    pltpu.sync_copy(zeros_hbm.at[pl.ds(r0, RPT)], acc.at[pl.ds(r0, RPT)])
    pltpu.sync_copy(ones_hbm, ones_v)
    plsc.subcore_barrier()

    def body(i, carry):
        @pl.when(i > 0)
        def _():
            pltpu.make_async_copy(ones_v, acc.at[eidx0.at[1]], ss0).wait()

        pltpu.sync_copy(edg_hbm.at[t, 2 * i], eidx0)
        pltpu.async_copy(ones_v, acc.at[eidx0.at[1]], ss0, add=True)

        @pl.when(i > 0)
        def _():
            pltpu.make_async_copy(ones_v, acc.at[eidx1.at[1]], ss1).wait()

        pltpu.sync_copy(edg_hbm.at[t, 2 * i + 1], eidx1)
        pltpu.async_copy(ones_v, acc.at[eidx1.at[1]], ss1, add=True)
        return carry

    lax.fori_loop(0, CH // 2, body, 0)
    pltpu.make_async_copy(ones_v, acc.at[eidx0.at[1]], ss0).wait()
    pltpu.make_async_copy(ones_v, acc.at[eidx1.at[1]], ss1).wait()
    plsc.subcore_barrier()
    pltpu.sync_copy(acc.at[pl.ds(r0, RPT)], out_hbm.at[c, pl.ds(r0, RPT)])


@functools.partial(
    pl.kernel,
    out_type=jax.ShapeDtypeStruct((2, NP, 128), jnp.float32),
    mesh=_mesh,
    scratch_types=[
        pltpu.VMEM((2, K), jnp.int32),
        pltpu.VMEM((2, K), jnp.int32),
        pltpu.VMEM((2, K), jnp.int32),
        pltpu.VMEM((K, 128), jnp.float32),
        pltpu.VMEM((K, 128), jnp.float32),
        pltpu.VMEM((K, 128), jnp.float32),
        pltpu.VMEM_SHARED((NP, 128), jnp.float32),
        pltpu.SemaphoreType.DMA,
        pltpu.SemaphoreType.DMA,
        pltpu.SemaphoreType.DMA,
        pltpu.SemaphoreType.DMA,
        pltpu.SemaphoreType.DMA,
        pltpu.SemaphoreType.DMA,
    ],
)
def _conv_kernel(hs_hbm, edg_hbm, zeros_hbm, out_hbm,
                 eidx0, eidx1, eidx2, rows0, rows1, rows2, acc,
                 gs0, gs1, gs2, ss0, ss1, ss2):
    c = lax.axis_index("c")
    s = lax.axis_index("s")
    t = c * 16 + s
    r0 = s * RPT
    pltpu.sync_copy(zeros_hbm.at[pl.ds(r0, RPT)], acc.at[pl.ds(r0, RPT)])
    plsc.subcore_barrier()

    eidx = (eidx0, eidx1, eidx2)
    rows = (rows0, rows1, rows2)
    gs = (gs0, gs1, gs2)
    ss = (ss0, ss1, ss2)

    def body(i, carry):
        for k in range(3):
            @pl.when(i > 0)
            def _():
                pltpu.make_async_copy(rows[k], acc.at[eidx[k].at[1]],
                                      ss[k]).wait()

            pltpu.sync_copy(edg_hbm.at[t, 3 * i + k], eidx[k])
            pltpu.async_copy(hs_hbm.at[eidx[k].at[0]], rows[k], gs[k])

        for k in range(3):
            pltpu.make_async_copy(hs_hbm.at[eidx[k].at[0]], rows[k],
                                  gs[k]).wait()
            pltpu.async_copy(rows[k], acc.at[eidx[k].at[1]], ss[k], add=True)
        return carry

    lax.fori_loop(0, CH // 3, body, 0)
    for k in range(3):
        pltpu.make_async_copy(rows[k], acc.at[eidx[k].at[1]], ss[k]).wait()
    plsc.subcore_barrier()
    pltpu.sync_copy(acc.at[pl.ds(r0, RPT)], out_hbm.at[c, pl.ds(r0, RPT)])


def _tc_mm1_body(x_ref, w1_ref, xw_ref):
    xw_ref[...] = jnp.dot(x_ref[...], w1_ref[...],
                          preferred_element_type=jnp.float32)


def _tc_pre_body(xw_ref, degp_ref, h1s_ref, dinv_ref):
    total = degp_ref[0] + degp_ref[1] + 1.0
    dinvb = jax.lax.rsqrt(total)
    h1s_ref[...] = dinvb * xw_ref[...]
    dinv_ref[...] = dinvb


def _tc_mid_body(a_ref, h1s_ref, dinv_ref, b1_ref, g_ref, bb_ref, w2_ref,
                 out_ref):
    dinv = dinv_ref[...]
    t = dinv * (a_ref[0] + a_ref[1] + h1s_ref[...]) + b1_ref[...]
    r = jnp.maximum(t, 0.0)
    mu = jnp.mean(r, axis=1, keepdims=True)
    d = r - mu
    var = jnp.mean(d * d, axis=1, keepdims=True)
    ln = d * jax.lax.rsqrt(var + 1e-5) * g_ref[...] + bb_ref[...]
    h2 = jnp.dot(ln, w2_ref[...], preferred_element_type=jnp.float32)
    out_ref[...] = dinv * h2


def _tc_post_body(a_ref, h2s_ref, dinv_ref, b2_ref, batch_ref,
                  w3_ref, b3_ref, w4_ref, b4_ref,
                  emb_ref, out2_ref, pooled):
    i = pl.program_id(0)
    emb = dinv_ref[...] * (a_ref[0] + a_ref[1] + h2s_ref[...]) + b2_ref[...]
    emb_ref[...] = emb
    h = jnp.maximum(emb, 0.0)
    bt = batch_ref[0, 0]
    onehot = (bt[:, None] == lax.broadcasted_iota(jnp.int32, (BLK, G), 1)
              ).astype(jnp.float32)
    contrib = lax.dot_general(onehot, h, (((0,), (0,)), ((), ())),
                              preferred_element_type=jnp.float32)

    @pl.when(i == 0)
    def _():
        pooled[...] = contrib

    @pl.when(i > 0)
    def _():
        pooled[...] = pooled[...] + contrib

    @pl.when(i == NB - 1)
    def _():
        z = jnp.dot(pooled[...], w3_ref[...],
                    preferred_element_type=jnp.float32) + b3_ref[...]
        z2 = jnp.dot(z, w4_ref[...],
                     preferred_element_type=jnp.float32) + b4_ref[...]
        col = lax.broadcasted_iota(jnp.int32, (G, 128), 1)
        z2m = jnp.where(col < 10, z2, -jnp.inf)
        m = jnp.max(z2m, axis=1, keepdims=True)
        lse = jnp.log(jnp.sum(jnp.exp(z2m - m), axis=1, keepdims=True)) + m
        out2_ref[...] = z2m - lse


_full = lambda shape: pl.BlockSpec(shape, lambda i: tuple(0 for _ in shape))
_rows = lambda shape: pl.BlockSpec(shape, lambda i: (i,) + tuple(
    0 for _ in shape[1:]))
_parts = lambda w: pl.BlockSpec((2, BLK, w), lambda i: (0, i, 0))

_tc_mm1 = pl.pallas_call(
    _tc_mm1_body,
    grid=(NB,),
    in_specs=[_rows((BLK, D)), _full((D, D))],
    out_specs=_rows((BLK, D)),
    out_shape=jax.ShapeDtypeStruct((NP, D), jnp.float32),
)

_tc_pre = pl.pallas_call(
    _tc_pre_body,
    grid=(NB,),
    in_specs=[_rows((BLK, D)), _parts(D)],
    out_specs=[_rows((BLK, D)), _rows((BLK, D))],
    out_shape=[jax.ShapeDtypeStruct((NP, D), jnp.float32),
               jax.ShapeDtypeStruct((NP, D), jnp.float32)],
)

_tc_mid = pl.pallas_call(
    _tc_mid_body,
    grid=(NB,),
    in_specs=[_parts(D), _rows((BLK, D)), _rows((BLK, D)),
              _full((1, D)), _full((1, D)), _full((1, D)), _full((D, D))],
    out_specs=_rows((BLK, D)),
    out_shape=jax.ShapeDtypeStruct((NP, D), jnp.float32),
)

_tc_post = pl.pallas_call(
    _tc_post_body,
    grid=(NB,),
    in_specs=[_parts(D), _rows((BLK, D)), _rows((BLK, D)), _full((1, D)),
              pl.BlockSpec((1, 1, BLK), lambda i: (i, 0, 0)),
              _full((D, D)), _full((1, D)), _full((D, 128)), _full((1, 128))],
    out_specs=[_rows((BLK, D)), _full((G, 128))],
    out_shape=[jax.ShapeDtypeStruct((NP, D), jnp.float32),
               jax.ShapeDtypeStruct((G, 128), jnp.float32)],
    scratch_shapes=[pltpu.VMEM((G, 128), jnp.float32)],
)


@jax.jit
def kernel(x, edge_index, batch, W1, b1, W2, b2, ln_g, ln_b, W3, b3, W4, b4):
    src = edge_index[0]
    dst = edge_index[1]
    pad = EP - E
    srcp = jnp.concatenate([src, jnp.zeros((pad,), jnp.int32)])
    dstp = jnp.concatenate([dst, jnp.full((pad,), N, jnp.int32)])
    edg = jnp.stack([srcp.reshape(NTILES, CH, K),
                     dstp.reshape(NTILES, CH, K)], axis=2)

    xp = jnp.pad(x, ((0, NP - N), (0, 0)))
    batch_p = jnp.concatenate([batch, jnp.full((NP - N,), G, jnp.int32)])
    batch2 = batch_p.reshape(NB, 1, BLK)

    zeros128 = jnp.zeros((NP, 128), jnp.float32)

    ones128 = jnp.ones((K, 128), jnp.float32)
    degf = _deg_kernel(edg, ones128, zeros128)
    xw = _tc_mm1(xp, W1)

    h1s, dinv_b = _tc_pre(xw, degf)
    acc1 = _conv_kernel(h1s, edg, zeros128)
    h2s = _tc_mid(acc1, h1s, dinv_b, b1.reshape(1, D), ln_g.reshape(1, D),
                  ln_b.reshape(1, D), W2)
    acc2 = _conv_kernel(h2s, edg, zeros128)

    W4p = jnp.pad(W4, ((0, 0), (0, 128 - W4.shape[1])))
    b4p = jnp.pad(b4, (0, 128 - b4.shape[0])).reshape(1, 128)
    emb_full, out2 = _tc_post(acc2, h2s, dinv_b, b2.reshape(1, D), batch2,
                              W3, b3.reshape(1, D), W4p, b4p)
    return emb_full[:N], out2[:, :10]


# speedup vs baseline: 1.0417x; 1.0417x over previous
import functools

import jax
import jax.numpy as jnp
from jax import lax
from jax.experimental import pallas as pl
from jax.experimental.pallas import tpu as pltpu
from jax.experimental.pallas import tpu_sc as plsc

N = 10000
E = 320000
D = 128
G = 64

NTILES = 32
K = 120
CH = 84
EP = NTILES * CH * K
NP = 10240
RPT = NP // 16
BLK = 256
NB = NP // BLK

_mesh = plsc.VectorSubcoreMesh(core_axis_name="c", subcore_axis_name="s")


@functools.partial(
    pl.kernel,
    out_type=jax.ShapeDtypeStruct((2, NP, 128), jnp.float32),
    mesh=_mesh,
    scratch_types=[
        pltpu.VMEM((2, K), jnp.int32),
        pltpu.VMEM((2, K), jnp.int32),
        pltpu.VMEM((K, 128), jnp.float32),
        pltpu.VMEM_SHARED((NP, 128), jnp.float32),
        pltpu.SemaphoreType.DMA,
        pltpu.SemaphoreType.DMA,
    ],
)
def _deg_kernel(edg_hbm, ones_hbm, zeros_hbm, out_hbm,
                eidx0, eidx1, ones_v, acc, ss0, ss1):
    c = lax.axis_index("c")
    s = lax.axis_index("s")
    t = c * 16 + s
    r0 = s * RPT
    pltpu.sync_copy(zeros_hbm.at[pl.ds(r0, RPT)], acc.at[pl.ds(r0, RPT)])
    pltpu.sync_copy(ones_hbm, ones_v)
    plsc.subcore_barrier()

    def body(i, carry):
        @pl.when(i > 0)
        def _():
            pltpu.make_async_copy(ones_v, acc.at[eidx0.at[1]], ss0).wait()

        pltpu.sync_copy(edg_hbm.at[t, 2 * i], eidx0)
        pltpu.async_copy(ones_v, acc.at[eidx0.at[1]], ss0, add=True)

        @pl.when(i > 0)
        def _():
            pltpu.make_async_copy(ones_v, acc.at[eidx1.at[1]], ss1).wait()

        pltpu.sync_copy(edg_hbm.at[t, 2 * i + 1], eidx1)
        pltpu.async_copy(ones_v, acc.at[eidx1.at[1]], ss1, add=True)
        return carry

    lax.fori_loop(0, CH // 2, body, 0)
    pltpu.make_async_copy(ones_v, acc.at[eidx0.at[1]], ss0).wait()
    pltpu.make_async_copy(ones_v, acc.at[eidx1.at[1]], ss1).wait()
    plsc.subcore_barrier()
    pltpu.sync_copy(acc.at[pl.ds(r0, RPT)], out_hbm.at[c, pl.ds(r0, RPT)])


@functools.partial(
    pl.kernel,
    out_type=jax.ShapeDtypeStruct((2, NP, 128), jnp.float32),
    mesh=_mesh,
    scratch_types=[
        pltpu.VMEM((2, 2, K), jnp.int32),
        pltpu.VMEM((2, 2, K), jnp.int32),
        pltpu.VMEM((2, 2, K), jnp.int32),
        pltpu.VMEM((K, 128), jnp.float32),
        pltpu.VMEM((K, 128), jnp.float32),
        pltpu.VMEM((K, 128), jnp.float32),
        pltpu.VMEM_SHARED((NP, 128), jnp.float32),
        pltpu.SemaphoreType.DMA,
        pltpu.SemaphoreType.DMA,
        pltpu.SemaphoreType.DMA,
        pltpu.SemaphoreType.DMA,
        pltpu.SemaphoreType.DMA,
        pltpu.SemaphoreType.DMA,
        pltpu.SemaphoreType.DMA,
        pltpu.SemaphoreType.DMA,
        pltpu.SemaphoreType.DMA,
        pltpu.SemaphoreType.DMA,
        pltpu.SemaphoreType.DMA,
        pltpu.SemaphoreType.DMA,
    ],
)
def _conv_kernel(hs_hbm, edg_hbm, zeros_hbm, out_hbm,
                 eidx0, eidx1, eidx2, rows0, rows1, rows2, acc,
                 gs0, gs1, gs2, ss0, ss1, ss2,
                 is0a, is1a, is2a, is0b, is1b, is2b):
    c = lax.axis_index("c")
    s = lax.axis_index("s")
    t = c * 16 + s
    r0 = s * RPT
    pltpu.sync_copy(zeros_hbm.at[pl.ds(r0, RPT)], acc.at[pl.ds(r0, RPT)])
    plsc.subcore_barrier()

    eidx = (eidx0, eidx1, eidx2)
    rows = (rows0, rows1, rows2)
    gs = (gs0, gs1, gs2)
    ss = (ss0, ss1, ss2)
    isem = ((is0a, is1a, is2a), (is0b, is1b, is2b))
    NI = CH // 3

    for k in range(3):
        pltpu.async_copy(edg_hbm.at[t, k], eidx[k].at[0], isem[0][k])

    def phase(i, p, q):
        for k in range(3):
            @pl.when(i > 0)
            def _():
                pltpu.make_async_copy(rows[k], acc.at[eidx[k].at[q, 1]],
                                      ss[k]).wait()

            @pl.when(i + 1 < NI)
            def _():
                pltpu.async_copy(edg_hbm.at[t, 3 * (i + 1) + k],
                                 eidx[k].at[q], isem[q][k])

            pltpu.make_async_copy(edg_hbm.at[t, 3 * i + k],
                                  eidx[k].at[p], isem[p][k]).wait()
            pltpu.async_copy(hs_hbm.at[eidx[k].at[p, 0]], rows[k], gs[k])

        for k in range(3):
            pltpu.make_async_copy(hs_hbm.at[eidx[k].at[p, 0]], rows[k],
                                  gs[k]).wait()
            pltpu.async_copy(rows[k], acc.at[eidx[k].at[p, 1]], ss[k],
                             add=True)

    def body(m, carry):
        phase(2 * m, 0, 1)
        phase(2 * m + 1, 1, 0)
        return carry

    lax.fori_loop(0, NI // 2, body, 0)
    for k in range(3):
        pltpu.make_async_copy(rows[k], acc.at[eidx[k].at[1, 1]], ss[k]).wait()
    plsc.subcore_barrier()
    pltpu.sync_copy(acc.at[pl.ds(r0, RPT)], out_hbm.at[c, pl.ds(r0, RPT)])


def _tc_pre_body(x_ref, w1_ref, degp_ref, h1s_ref, dinv_ref):
    total = degp_ref[0] + degp_ref[1] + 1.0
    dinvb = jax.lax.rsqrt(total)
    h = jnp.dot(x_ref[...], w1_ref[...], preferred_element_type=jnp.float32)
    h1s_ref[...] = dinvb * h
    dinv_ref[...] = dinvb


def _tc_mid_body(a_ref, h1s_ref, dinv_ref, b1_ref, g_ref, bb_ref, w2_ref,
                 out_ref):
    dinv = dinv_ref[...]
    t = dinv * (a_ref[0] + a_ref[1] + h1s_ref[...]) + b1_ref[...]
    r = jnp.maximum(t, 0.0)
    mu = jnp.mean(r, axis=1, keepdims=True)
    d = r - mu
    var = jnp.mean(d * d, axis=1, keepdims=True)
    ln = d * jax.lax.rsqrt(var + 1e-5) * g_ref[...] + bb_ref[...]
    h2 = jnp.dot(ln, w2_ref[...], preferred_element_type=jnp.float32)
    out_ref[...] = dinv * h2


def _tc_post_body(a_ref, h2s_ref, dinv_ref, b2_ref, batch_ref,
                  w3_ref, b3_ref, w4_ref, b4_ref,
                  emb_ref, out2_ref, pooled):
    i = pl.program_id(0)
    emb = dinv_ref[...] * (a_ref[0] + a_ref[1] + h2s_ref[...]) + b2_ref[...]
    emb_ref[...] = emb
    h = jnp.maximum(emb, 0.0)
    bt = batch_ref[0, 0]
    onehot = (bt[:, None] == lax.broadcasted_iota(jnp.int32, (BLK, G), 1)
              ).astype(jnp.float32)
    contrib = lax.dot_general(onehot, h, (((0,), (0,)), ((), ())),
                              preferred_element_type=jnp.float32)

    @pl.when(i == 0)
    def _():
        pooled[...] = contrib

    @pl.when(i > 0)
    def _():
        pooled[...] = pooled[...] + contrib

    @pl.when(i == NB - 1)
    def _():
        z = jnp.dot(pooled[...], w3_ref[...],
                    preferred_element_type=jnp.float32) + b3_ref[...]
        z2 = jnp.dot(z, w4_ref[...],
                     preferred_element_type=jnp.float32) + b4_ref[...]
        col = lax.broadcasted_iota(jnp.int32, (G, 128), 1)
        z2m = jnp.where(col < 10, z2, -jnp.inf)
        m = jnp.max(z2m, axis=1, keepdims=True)
        lse = jnp.log(jnp.sum(jnp.exp(z2m - m), axis=1, keepdims=True)) + m
        out2_ref[...] = z2m - lse


_full = lambda shape: pl.BlockSpec(shape, lambda i: tuple(0 for _ in shape))
_rows = lambda shape: pl.BlockSpec(shape, lambda i: (i,) + tuple(
    0 for _ in shape[1:]))
_parts = lambda w: pl.BlockSpec((2, BLK, w), lambda i: (0, i, 0))

_tc_pre = pl.pallas_call(
    _tc_pre_body,
    grid=(NB,),
    in_specs=[_rows((BLK, D)), _full((D, D)), _parts(D)],
    out_specs=[_rows((BLK, D)), _rows((BLK, D))],
    out_shape=[jax.ShapeDtypeStruct((NP, D), jnp.float32),
               jax.ShapeDtypeStruct((NP, D), jnp.float32)],
)

_tc_mid = pl.pallas_call(
    _tc_mid_body,
    grid=(NB,),
    in_specs=[_parts(D), _rows((BLK, D)), _rows((BLK, D)),
              _full((1, D)), _full((1, D)), _full((1, D)), _full((D, D))],
    out_specs=_rows((BLK, D)),
    out_shape=jax.ShapeDtypeStruct((NP, D), jnp.float32),
)

_tc_post = pl.pallas_call(
    _tc_post_body,
    grid=(NB,),
    in_specs=[_parts(D), _rows((BLK, D)), _rows((BLK, D)), _full((1, D)),
              pl.BlockSpec((1, 1, BLK), lambda i: (i, 0, 0)),
              _full((D, D)), _full((1, D)), _full((D, 128)), _full((1, 128))],
    out_specs=[_rows((BLK, D)), _full((G, 128))],
    out_shape=[jax.ShapeDtypeStruct((NP, D), jnp.float32),
               jax.ShapeDtypeStruct((G, 128), jnp.float32)],
    scratch_shapes=[pltpu.VMEM((G, 128), jnp.float32)],
)


@jax.jit
def kernel(x, edge_index, batch, W1, b1, W2, b2, ln_g, ln_b, W3, b3, W4, b4):
    src = edge_index[0]
    dst = edge_index[1]
    pad = EP - E
    srcp = jnp.concatenate([src, jnp.zeros((pad,), jnp.int32)])
    dstp = jnp.concatenate([dst, jnp.full((pad,), N, jnp.int32)])
    edg = jnp.stack([srcp.reshape(NTILES, CH, K),
                     dstp.reshape(NTILES, CH, K)], axis=2)

    xp = jnp.pad(x, ((0, NP - N), (0, 0)))
    batch_p = jnp.concatenate([batch, jnp.full((NP - N,), G, jnp.int32)])
    batch2 = batch_p.reshape(NB, 1, BLK)

    zeros128 = jnp.zeros((NP, 128), jnp.float32)

    ones128 = jnp.ones((K, 128), jnp.float32)
    degf = _deg_kernel(edg, ones128, zeros128)

    h1s, dinv_b = _tc_pre(xp, W1, degf)
    acc1 = _conv_kernel(h1s, edg, zeros128)
    h2s = _tc_mid(acc1, h1s, dinv_b, b1.reshape(1, D), ln_g.reshape(1, D),
                  ln_b.reshape(1, D), W2)
    acc2 = _conv_kernel(h2s, edg, zeros128)

    W4p = jnp.pad(W4, ((0, 0), (0, 128 - W4.shape[1])))
    b4p = jnp.pad(b4, (0, 128 - b4.shape[0])).reshape(1, 128)
    emb_full, out2 = _tc_post(acc2, h2s, dinv_b, b2.reshape(1, D), batch2,
                              W3, b3.reshape(1, D), W4p, b4p)
    return emb_full[:N], out2[:, :10]


# speedup vs baseline: 1.0443x; 1.0024x over previous
import functools

import jax
import jax.numpy as jnp
from jax import lax
from jax.experimental import pallas as pl
from jax.experimental.pallas import tpu as pltpu
from jax.experimental.pallas import tpu_sc as plsc

N = 10000
E = 320000
D = 128
G = 64

NTILES = 32
K = 120
CH = 84
EP = NTILES * CH * K
NP = 10240
RPT = NP // 16
BLK = 256
NB = NP // BLK

_mesh = plsc.VectorSubcoreMesh(core_axis_name="c", subcore_axis_name="s")


@functools.partial(
    pl.kernel,
    out_type=jax.ShapeDtypeStruct((2, NP, 128), jnp.float32),
    mesh=_mesh,
    scratch_types=[
        pltpu.VMEM((2, 2, K), jnp.int32),
        pltpu.VMEM((2, 2, K), jnp.int32),
        pltpu.VMEM((K, 128), jnp.float32),
        pltpu.VMEM_SHARED((NP, 128), jnp.float32),
        pltpu.SemaphoreType.DMA,
        pltpu.SemaphoreType.DMA,
        pltpu.SemaphoreType.DMA,
        pltpu.SemaphoreType.DMA,
        pltpu.SemaphoreType.DMA,
        pltpu.SemaphoreType.DMA,
    ],
)
def _deg_kernel(edg_hbm, ones_hbm, zeros_hbm, out_hbm,
                eidx0, eidx1, ones_v, acc,
                ss0, ss1, is0a, is1a, is0b, is1b):
    c = lax.axis_index("c")
    s = lax.axis_index("s")
    t = c * 16 + s
    r0 = s * RPT
    pltpu.sync_copy(zeros_hbm.at[pl.ds(r0, RPT)], acc.at[pl.ds(r0, RPT)])
    pltpu.sync_copy(ones_hbm, ones_v)
    plsc.subcore_barrier()

    eidx = (eidx0, eidx1)
    ss = (ss0, ss1)
    isem = ((is0a, is1a), (is0b, is1b))
    NI = CH // 2

    for k in range(2):
        pltpu.async_copy(edg_hbm.at[t, k], eidx[k].at[0], isem[0][k])

    def phase(i, p, q):
        for k in range(2):
            @pl.when(i > 0)
            def _():
                pltpu.make_async_copy(ones_v, acc.at[eidx[k].at[q, 1]],
                                      ss[k]).wait()

            @pl.when(i + 1 < NI)
            def _():
                pltpu.async_copy(edg_hbm.at[t, 2 * (i + 1) + k],
                                 eidx[k].at[q], isem[q][k])

            pltpu.make_async_copy(edg_hbm.at[t, 2 * i + k],
                                  eidx[k].at[p], isem[p][k]).wait()
            pltpu.async_copy(ones_v, acc.at[eidx[k].at[p, 1]], ss[k],
                             add=True)

    def body(m, carry):
        phase(2 * m, 0, 1)
        phase(2 * m + 1, 1, 0)
        return carry

    lax.fori_loop(0, NI // 2, body, 0)
    for k in range(2):
        pltpu.make_async_copy(ones_v, acc.at[eidx[k].at[1, 1]], ss[k]).wait()
    plsc.subcore_barrier()
    pltpu.sync_copy(acc.at[pl.ds(r0, RPT)], out_hbm.at[c, pl.ds(r0, RPT)])


@functools.partial(
    pl.kernel,
    out_type=jax.ShapeDtypeStruct((2, NP, 128), jnp.float32),
    mesh=_mesh,
    scratch_types=[
        pltpu.VMEM((2, 2, K), jnp.int32),
        pltpu.VMEM((2, 2, K), jnp.int32),
        pltpu.VMEM((2, 2, K), jnp.int32),
        pltpu.VMEM((K, 128), jnp.float32),
        pltpu.VMEM((K, 128), jnp.float32),
        pltpu.VMEM((K, 128), jnp.float32),
        pltpu.VMEM_SHARED((NP, 128), jnp.float32),
        pltpu.SemaphoreType.DMA,
        pltpu.SemaphoreType.DMA,
        pltpu.SemaphoreType.DMA,
        pltpu.SemaphoreType.DMA,
        pltpu.SemaphoreType.DMA,
        pltpu.SemaphoreType.DMA,
        pltpu.SemaphoreType.DMA,
        pltpu.SemaphoreType.DMA,
        pltpu.SemaphoreType.DMA,
        pltpu.SemaphoreType.DMA,
        pltpu.SemaphoreType.DMA,
        pltpu.SemaphoreType.DMA,
    ],
)
def _conv_kernel(hs_hbm, edg_hbm, zeros_hbm, out_hbm,
                 eidx0, eidx1, eidx2, rows0, rows1, rows2, acc,
                 gs0, gs1, gs2, ss0, ss1, ss2,
                 is0a, is1a, is2a, is0b, is1b, is2b):
    c = lax.axis_index("c")
    s = lax.axis_index("s")
    t = c * 16 + s
    r0 = s * RPT
    pltpu.sync_copy(zeros_hbm.at[pl.ds(r0, RPT)], acc.at[pl.ds(r0, RPT)])
    plsc.subcore_barrier()

    eidx = (eidx0, eidx1, eidx2)
    rows = (rows0, rows1, rows2)
    gs = (gs0, gs1, gs2)
    ss = (ss0, ss1, ss2)
    isem = ((is0a, is1a, is2a), (is0b, is1b, is2b))
    NI = CH // 3

    for k in range(3):
        pltpu.async_copy(edg_hbm.at[t, k], eidx[k].at[0], isem[0][k])

    def phase(i, p, q):
        for k in range(3):
            @pl.when(i > 0)
            def _():
                pltpu.make_async_copy(rows[k], acc.at[eidx[k].at[q, 1]],
                                      ss[k]).wait()

            @pl.when(i + 1 < NI)
            def _():
                pltpu.async_copy(edg_hbm.at[t, 3 * (i + 1) + k],
                                 eidx[k].at[q], isem[q][k])

            pltpu.make_async_copy(edg_hbm.at[t, 3 * i + k],
                                  eidx[k].at[p], isem[p][k]).wait()
            pltpu.async_copy(hs_hbm.at[eidx[k].at[p, 0]], rows[k], gs[k])

        for k in range(3):
            pltpu.make_async_copy(hs_hbm.at[eidx[k].at[p, 0]], rows[k],
                                  gs[k]).wait()
            pltpu.async_copy(rows[k], acc.at[eidx[k].at[p, 1]], ss[k],
                             add=True)

    def body(m, carry):
        phase(2 * m, 0, 1)
        phase(2 * m + 1, 1, 0)
        return carry

    lax.fori_loop(0, NI // 2, body, 0)
    for k in range(3):
        pltpu.make_async_copy(rows[k], acc.at[eidx[k].at[1, 1]], ss[k]).wait()
    plsc.subcore_barrier()
    pltpu.sync_copy(acc.at[pl.ds(r0, RPT)], out_hbm.at[c, pl.ds(r0, RPT)])


def _tc_pre_body(x_ref, w1_ref, degp_ref, h1s_ref, dinv_ref):
    total = degp_ref[0] + degp_ref[1] + 1.0
    dinvb = jax.lax.rsqrt(total)
    h = jnp.dot(x_ref[...], w1_ref[...], preferred_element_type=jnp.float32)
    h1s_ref[...] = dinvb * h
    dinv_ref[...] = dinvb


def _tc_mid_body(a_ref, h1s_ref, dinv_ref, b1_ref, g_ref, bb_ref, w2_ref,
                 out_ref):
    dinv = dinv_ref[...]
    t = dinv * (a_ref[0] + a_ref[1] + h1s_ref[...]) + b1_ref[...]
    r = jnp.maximum(t, 0.0)
    mu = jnp.mean(r, axis=1, keepdims=True)
    d = r - mu
    var = jnp.mean(d * d, axis=1, keepdims=True)
    ln = d * jax.lax.rsqrt(var + 1e-5) * g_ref[...] + bb_ref[...]
    h2 = jnp.dot(ln, w2_ref[...], preferred_element_type=jnp.float32)
    out_ref[...] = dinv * h2


def _tc_post_body(a_ref, h2s_ref, dinv_ref, b2_ref, batch_ref,
                  w3_ref, b3_ref, w4_ref, b4_ref,
                  emb_ref, out2_ref, pooled):
    i = pl.program_id(0)
    emb = dinv_ref[...] * (a_ref[0] + a_ref[1] + h2s_ref[...]) + b2_ref[...]
    emb_ref[...] = emb
    h = jnp.maximum(emb, 0.0)
    bt = batch_ref[0, 0]
    onehot = (bt[:, None] == lax.broadcasted_iota(jnp.int32, (BLK, G), 1)
              ).astype(jnp.float32)
    contrib = lax.dot_general(onehot, h, (((0,), (0,)), ((), ())),
                              preferred_element_type=jnp.float32)

    @pl.when(i == 0)
    def _():
        pooled[...] = contrib

    @pl.when(i > 0)
    def _():
        pooled[...] = pooled[...] + contrib

    @pl.when(i == NB - 1)
    def _():
        z = jnp.dot(pooled[...], w3_ref[...],
                    preferred_element_type=jnp.float32) + b3_ref[...]
        z2 = jnp.dot(z, w4_ref[...],
                     preferred_element_type=jnp.float32) + b4_ref[...]
        col = lax.broadcasted_iota(jnp.int32, (G, 128), 1)
        z2m = jnp.where(col < 10, z2, -jnp.inf)
        m = jnp.max(z2m, axis=1, keepdims=True)
        lse = jnp.log(jnp.sum(jnp.exp(z2m - m), axis=1, keepdims=True)) + m
        out2_ref[...] = z2m - lse


_full = lambda shape: pl.BlockSpec(shape, lambda i: tuple(0 for _ in shape))
_rows = lambda shape: pl.BlockSpec(shape, lambda i: (i,) + tuple(
    0 for _ in shape[1:]))
_parts = lambda w: pl.BlockSpec((2, BLK, w), lambda i: (0, i, 0))

_tc_pre = pl.pallas_call(
    _tc_pre_body,
    grid=(NB,),
    in_specs=[_rows((BLK, D)), _full((D, D)), _parts(D)],
    out_specs=[_rows((BLK, D)), _rows((BLK, D))],
    out_shape=[jax.ShapeDtypeStruct((NP, D), jnp.float32),
               jax.ShapeDtypeStruct((NP, D), jnp.float32)],
)

_tc_mid = pl.pallas_call(
    _tc_mid_body,
    grid=(NB,),
    in_specs=[_parts(D), _rows((BLK, D)), _rows((BLK, D)),
              _full((1, D)), _full((1, D)), _full((1, D)), _full((D, D))],
    out_specs=_rows((BLK, D)),
    out_shape=jax.ShapeDtypeStruct((NP, D), jnp.float32),
)

_tc_post = pl.pallas_call(
    _tc_post_body,
    grid=(NB,),
    in_specs=[_parts(D), _rows((BLK, D)), _rows((BLK, D)), _full((1, D)),
              pl.BlockSpec((1, 1, BLK), lambda i: (i, 0, 0)),
              _full((D, D)), _full((1, D)), _full((D, 128)), _full((1, 128))],
    out_specs=[_rows((BLK, D)), _full((G, 128))],
    out_shape=[jax.ShapeDtypeStruct((NP, D), jnp.float32),
               jax.ShapeDtypeStruct((G, 128), jnp.float32)],
    scratch_shapes=[pltpu.VMEM((G, 128), jnp.float32)],
)


@jax.jit
def kernel(x, edge_index, batch, W1, b1, W2, b2, ln_g, ln_b, W3, b3, W4, b4):
    src = edge_index[0]
    dst = edge_index[1]
    pad = EP - E
    srcp = jnp.concatenate([src, jnp.zeros((pad,), jnp.int32)])
    dstp = jnp.concatenate([dst, jnp.full((pad,), N, jnp.int32)])
    edg = jnp.stack([srcp.reshape(NTILES, CH, K),
                     dstp.reshape(NTILES, CH, K)], axis=2)

    xp = jnp.pad(x, ((0, NP - N), (0, 0)))
    batch_p = jnp.concatenate([batch, jnp.full((NP - N,), G, jnp.int32)])
    batch2 = batch_p.reshape(NB, 1, BLK)

    zeros128 = jnp.zeros((NP, 128), jnp.float32)

    ones128 = jnp.ones((K, 128), jnp.float32)
    degf = _deg_kernel(edg, ones128, zeros128)

    h1s, dinv_b = _tc_pre(xp, W1, degf)
    acc1 = _conv_kernel(h1s, edg, zeros128)
    h2s = _tc_mid(acc1, h1s, dinv_b, b1.reshape(1, D), ln_g.reshape(1, D),
                  ln_b.reshape(1, D), W2)
    acc2 = _conv_kernel(h2s, edg, zeros128)

    W4p = jnp.pad(W4, ((0, 0), (0, 128 - W4.shape[1])))
    b4p = jnp.pad(b4, (0, 128 - b4.shape[0])).reshape(1, 128)
    emb_full, out2 = _tc_post(acc2, h2s, dinv_b, b2.reshape(1, D), batch2,
                              W3, b3.reshape(1, D), W4p, b4p)
    return emb_full[:N], out2[:, :10]
